# baseline (speedup 1.0000x reference)
import functools

import jax
import jax.numpy as jnp
from jax import lax
from jax.experimental import pallas as pl
from jax.experimental.pallas import tpu as pltpu
from jax.experimental.pallas import tpu_sc as plsc


def _gather_last_nodes(features, n_node):
    B = n_node.shape[0]
    D = features.shape[1]
    mesh = plsc.ScalarSubcoreMesh(axis_name="c", num_cores=1)

    @functools.partial(
        pl.kernel,
        out_type=jax.ShapeDtypeStruct((B, D), features.dtype),
        scratch_types=[
            pltpu.SMEM((B,), jnp.int32),
            pltpu.SemaphoreType.DMA,
        ],
        mesh=mesh,
    )
    def body(features_hbm, n_node_hbm, out_hbm, nn_s, sem):
        pltpu.sync_copy(n_node_hbm, nn_s)

        run = jnp.int32(0)
        for g in range(B):
            run = run + nn_s[g]
            pltpu.make_async_copy(
                features_hbm.at[pl.ds(run - 1, 1)],
                out_hbm.at[pl.ds(g, 1)],
                sem,
            ).start()
        pltpu.make_async_copy(
            features_hbm.at[pl.ds(0, B)], out_hbm, sem
        ).wait()

    return body(features, n_node)


def kernel(features, n_node, n_edge, globals, edges, senders, receivers):
    n_node = jnp.reshape(n_node, (-1,)).astype(jnp.int32)
    return _gather_last_nodes(features, n_node)


# speedup vs baseline: 1.0238x; 1.0238x over previous
import functools

import jax
import jax.numpy as jnp
from jax import lax
from jax.experimental import pallas as pl
from jax.experimental.pallas import tpu as pltpu
from jax.experimental.pallas import tpu_sc as plsc


def _gather_last_nodes(features, n_node):
    B = n_node.shape[0]
    D = features.shape[1]
    mesh = plsc.ScalarSubcoreMesh(axis_name="c", num_cores=1)

    @functools.partial(
        pl.kernel,
        out_type=jax.ShapeDtypeStruct((B, D), features.dtype),
        scratch_types=[
            pltpu.SMEM((B,), jnp.int32),
            pltpu.SemaphoreType.DMA,
        ],
        mesh=mesh,
    )
    def body(features_hbm, n_node_hbm, out_hbm, nn_s, sem):
        pltpu.sync_copy(n_node_hbm, nn_s)

        def loop_body(g, run):
            run = run + nn_s[g]
            pltpu.make_async_copy(
                features_hbm.at[pl.ds(run - 1, 1)],
                out_hbm.at[pl.ds(g, 1)],
                sem,
            ).start()
            return run

        lax.fori_loop(0, B, loop_body, jnp.int32(0), unroll=8)
        pltpu.make_async_copy(
            features_hbm.at[pl.ds(0, B)], out_hbm, sem
        ).wait()

    return body(features, n_node)


def kernel(features, n_node, n_edge, globals, edges, senders, receivers):
    n_node = jnp.reshape(n_node, (-1,)).astype(jnp.int32)
    return _gather_last_nodes(features, n_node)


# speedup vs baseline: 1.0334x; 1.0093x over previous
import functools

import jax
import jax.numpy as jnp
from jax import lax
from jax.experimental import pallas as pl
from jax.experimental.pallas import tpu as pltpu
from jax.experimental.pallas import tpu_sc as plsc


def _gather_last_nodes(features, n_node):
    B = n_node.shape[0]
    D = features.shape[1]
    mesh = plsc.ScalarSubcoreMesh(axis_name="c", num_cores=1)

    @functools.partial(
        pl.kernel,
        out_type=jax.ShapeDtypeStruct((B, D), features.dtype),
        scratch_types=[
            pltpu.SMEM((B,), jnp.int32),
            pltpu.SemaphoreType.DMA,
        ],
        mesh=mesh,
    )
    def body(features_hbm, n_node_hbm, out_hbm, nn_s, sem):
        pltpu.sync_copy(n_node_hbm, nn_s)

        def loop_body(g, run):
            run = run + nn_s[g]
            pltpu.make_async_copy(
                features_hbm.at[pl.ds(run - 1, 1)],
                out_hbm.at[pl.ds(g, 1)],
                sem,
            ).start()
            return run

        lax.fori_loop(0, B, loop_body, jnp.int32(0))
        pltpu.make_async_copy(
            features_hbm.at[pl.ds(0, B)], out_hbm, sem
        ).wait()

    return body(features, n_node)


def kernel(features, n_node, n_edge, globals, edges, senders, receivers):
    n_node = jnp.reshape(n_node, (-1,)).astype(jnp.int32)
    return _gather_last_nodes(features, n_node)
